# baseline (speedup 1.0000x reference)
import jax
import jax.numpy as jnp
from jax.experimental import pallas as pl
from jax.experimental.pallas import tpu as pltpu


def _softmax_body(scale_ref, x_ref, o_ref):
    e = jnp.exp(x_ref[...] * scale_ref[0])
    o_ref[...] = e / jnp.sum(e, axis=-1, keepdims=True)


def kernel(branch, par, chi, t):
    x = branch[0]
    n, p = x.shape
    scale = (2.0 / jnp.asarray(t, jnp.float32)).reshape(1)
    block = 4096
    out = pl.pallas_call(
        _softmax_body,
        grid=(n // block,),
        in_specs=[
            pl.BlockSpec(memory_space=pltpu.SMEM),
            pl.BlockSpec((block, p), lambda i: (i, 0)),
        ],
        out_specs=pl.BlockSpec((block, p), lambda i: (i, 0)),
        out_shape=jax.ShapeDtypeStruct((n, p), jnp.float32),
    )(scale, x)
    return out


# speedup vs baseline: 1.3098x; 1.3098x over previous
import jax
import jax.numpy as jnp
from jax.experimental import pallas as pl
from jax.experimental.pallas import tpu as pltpu


def _softmax_body(scale_ref, x_ref, o_ref):
    e = jnp.exp(x_ref[0] * scale_ref[0])
    o_ref[...] = e / jnp.sum(e, axis=-1, keepdims=True)


def kernel(branch, par, chi, t):
    _, n, p = branch.shape
    scale = (2.0 / jnp.asarray(t, jnp.float32)).reshape(1)
    block = 4096
    out = pl.pallas_call(
        _softmax_body,
        grid=(n // block,),
        in_specs=[
            pl.BlockSpec(memory_space=pltpu.SMEM),
            pl.BlockSpec((1, block, p), lambda i: (0, i, 0)),
        ],
        out_specs=pl.BlockSpec((block, p), lambda i: (i, 0)),
        out_shape=jax.ShapeDtypeStruct((n, p), jnp.float32),
    )(scale, branch)
    return out
